# baseline (speedup 1.0000x reference)
import jax
import jax.numpy as jnp
from jax import lax
from jax.experimental import pallas as pl
from jax.experimental.pallas import tpu as pltpu
from jax.experimental.pallas import tpu_sc as plsc

K = 100000
D_OUT = 512
K_PAD = 100352
TBK = 4096
TNBLK = 18
KTC = TNBLK * TBK
KSC = K_PAD - KTC
RPW = KSC // 32
CH = 16
NPAIR = RPW // CH // 2
NW = 16
C = K_PAD // NW
VB = C // 16
TOPK = 10
NEG = -1e30
IBIG = 2147483647


def _tc_mlp_body(x_ref, w1_ref, b1_ref, w2_ref, b2_ref, s_ref, q_ref):
    h = jnp.maximum(
        jnp.dot(x_ref[...], w1_ref[...],
                preferred_element_type=jnp.float32) + b1_ref[...], 0.0)
    f = jnp.dot(h, w2_ref[...],
                preferred_element_type=jnp.float32) + b2_ref[...]
    nrm = jnp.maximum(jnp.sqrt(jnp.sum(f * f)), 1e-12)
    q_ref[...] = f * (jnp.exp(s_ref[0, 0]) / nrm)


def _tc_mlp(x, w1, b1, w2, b2, scale):
    return pl.pallas_call(
        _tc_mlp_body,
        out_shape=jax.ShapeDtypeStruct((1, D_OUT), jnp.float32),
    )(x, w1, b1, w2, b2, scale)


def _tc_mv_body(q_ref, loc_ref, out_ref):
    out_ref[...] = lax.dot_general(
        q_ref[...], loc_ref[...], (((1,), (1,)), ((), ())),
        preferred_element_type=jnp.float32)


def _tc_matvec(q, loc):
    return pl.pallas_call(
        _tc_mv_body,
        grid=(TNBLK,),
        in_specs=[
            pl.BlockSpec((1, D_OUT), lambda i: (0, 0)),
            pl.BlockSpec((TBK, D_OUT), lambda i: (i, 0)),
        ],
        out_specs=pl.BlockSpec((1, TBK), lambda i: (0, i)),
        out_shape=jax.ShapeDtypeStruct((1, KTC), jnp.float32),
        compiler_params=pltpu.CompilerParams(
            dimension_semantics=("arbitrary",)),
    )(q, loc)


def _sc_mv_body(q_hbm, loc_hbm, out_hbm, qv, r0, r1, ostage, sem0, sem1):
    cid = lax.axis_index("c")
    sid = lax.axis_index("s")
    w = sid * 2 + cid
    start = KTC + w * RPW
    iot = lax.iota(jnp.int32, 16)

    pltpu.sync_copy(q_hbm, qv)
    qs = [qv[pl.ds(16 * j, 16)] for j in range(32)]

    def row_off(g):
        return jnp.minimum(start + g * CH, K - CH)

    def compute_chunk(rbuf, g):
        o0 = jnp.zeros((16,), jnp.float32)
        for r in range(CH):
            acc = qs[0] * rbuf[r, pl.ds(0, 16)]
            for j in range(1, 32):
                acc = acc + qs[j] * rbuf[r, pl.ds(16 * j, 16)]
            o0 = jnp.where(iot == r, jnp.sum(acc), o0)
        ostage[pl.ds(g * CH, 16)] = o0

    pltpu.async_copy(loc_hbm.at[pl.ds(row_off(0), CH)], r0, sem0)
    pltpu.async_copy(loc_hbm.at[pl.ds(row_off(1), CH)], r1, sem1)

    def body(h, carry):
        g0 = 2 * h
        pltpu.make_async_copy(loc_hbm.at[pl.ds(0, CH)], r0, sem0).wait()
        compute_chunk(r0, g0)

        @pl.when(h < NPAIR - 1)
        def _():
            pltpu.async_copy(loc_hbm.at[pl.ds(row_off(g0 + 2), CH)], r0, sem0)

        pltpu.make_async_copy(loc_hbm.at[pl.ds(0, CH)], r1, sem1).wait()
        compute_chunk(r1, g0 + 1)

        @pl.when(h < NPAIR - 1)
        def _():
            pltpu.async_copy(loc_hbm.at[pl.ds(row_off(g0 + 3), CH)], r1, sem1)

        return carry

    lax.fori_loop(0, NPAIR, body, 0)
    pltpu.sync_copy(ostage, out_hbm.at[pl.ds(w * RPW, RPW)])


def _sc_matvec(q, loc):
    mesh = plsc.VectorSubcoreMesh(core_axis_name="c", subcore_axis_name="s")
    f32 = jnp.float32
    return pl.kernel(
        _sc_mv_body,
        out_type=jax.ShapeDtypeStruct((KSC,), f32),
        mesh=mesh,
        scratch_types=[
            pltpu.VMEM((D_OUT,), f32),
            pltpu.VMEM((CH, D_OUT), f32),
            pltpu.VMEM((CH, D_OUT), f32),
            pltpu.VMEM((RPW,), f32),
            pltpu.SemaphoreType.DMA,
            pltpu.SemaphoreType.DMA,
        ],
        compiler_params=pltpu.CompilerParams(
            needs_layout_passes=False, use_tc_tiling_on_sc=False),
    )(q, loc)


def _sc_body(logits_hbm, gps_hbm, out_gps_hbm, out_prob_hbm,
             buf, vals_buf, idx_buf, ms_buf,
             mvals, midx, mms, prob_buf, rows_v,
             sh_vals, sh_idx, sh_ms, sem):
    wid = lax.axis_index("s")
    base = wid * C
    iot = lax.iota(jnp.int32, 16)

    pltpu.sync_copy(logits_hbm.at[pl.ds(base, C)], buf)

    def p_mask(j, m_vec):
        v = buf[pl.ds(j * 16, 16)]
        gidx = base + j * 16 + iot
        v = jnp.where(gidx < K, v, NEG)
        buf[pl.ds(j * 16, 16)] = v
        return jnp.maximum(m_vec, v)

    m_vec = lax.fori_loop(0, VB, p_mask, jnp.full((16,), NEG, jnp.float32))
    m_w = jnp.max(m_vec)

    def p_sum(j, s_vec):
        v = buf[pl.ds(j * 16, 16)]
        return s_vec + jnp.exp(v - m_w)

    s_vec = lax.fori_loop(0, VB, p_sum, jnp.zeros((16,), jnp.float32))

    vals_vec = jnp.full((16,), NEG, jnp.float32)
    idx_vec = jnp.zeros((16,), jnp.int32)
    for i in range(TOPK):
        def p_top(j, carry):
            mx, mi = carry
            v = buf[pl.ds(j * 16, 16)]
            gidx = base + j * 16 + iot
            c = v > mx
            return jnp.where(c, v, mx), jnp.where(c, gidx, mi)

        mx, mi = lax.fori_loop(
            0, VB, p_top,
            (jnp.full((16,), NEG, jnp.float32), jnp.zeros((16,), jnp.int32)))
        gm = jnp.max(mx)
        gi = jnp.min(jnp.where(mx == gm, mi, IBIG))
        vals_vec = jnp.where(iot == i, gm, vals_vec)
        idx_vec = jnp.where(iot == i, gi, idx_vec)
        lo = gi - base
        j0 = lo & ~15
        vv = buf[pl.ds(j0, 16)]
        buf[pl.ds(j0, 16)] = jnp.where(iot == (lo & 15), NEG, vv)

    vals_buf[...] = vals_vec
    idx_buf[...] = idx_vec
    ms_buf[0, :] = jnp.broadcast_to(m_w, (16,))
    ms_buf[1, :] = s_vec

    pltpu.sync_copy(vals_buf, sh_vals.at[pl.ds(wid * 16, 16)])
    pltpu.sync_copy(idx_buf, sh_idx.at[pl.ds(wid * 16, 16)])
    pltpu.sync_copy(ms_buf, sh_ms.at[wid])
    plsc.subcore_barrier()

    @pl.when(wid == 0)
    def _merge():
        pltpu.sync_copy(sh_vals, mvals)
        pltpu.sync_copy(sh_idx, midx)
        pltpu.sync_copy(sh_ms, mms)

        m_all = jnp.full((16,), NEG, jnp.float32)
        for w in range(NW):
            m_all = jnp.maximum(m_all, mms[w, 0, :])
        s_all = jnp.zeros((16,), jnp.float32)
        for w in range(NW):
            s_all = s_all + mms[w, 1, :] * jnp.exp(mms[w, 0, :] - m_all)
        s_tot = jnp.sum(s_all)

        tvals = jnp.full((16,), NEG, jnp.float32)
        tidx = jnp.zeros((16,), jnp.int32)
        for i in range(TOPK):
            mx = jnp.full((16,), NEG, jnp.float32)
            gx = jnp.zeros((16,), jnp.int32)
            cp = jnp.zeros((16,), jnp.int32)
            for w in range(NW):
                v = mvals[pl.ds(w * 16, 16)]
                c = v > mx
                mx = jnp.where(c, v, mx)
                gx = jnp.where(c, midx[pl.ds(w * 16, 16)], gx)
                cp = jnp.where(c, w * 16 + iot, cp)
            gm = jnp.max(mx)
            gi = jnp.min(jnp.where(mx == gm, gx, IBIG))
            cpw = jnp.min(jnp.where((mx == gm) & (gx == gi), cp, IBIG))
            tvals = jnp.where(iot == i, gm, tvals)
            tidx = jnp.where(iot == i, gi, tidx)
            j0 = cpw & ~15
            vv = mvals[pl.ds(j0, 16)]
            mvals[pl.ds(j0, 16)] = jnp.where(iot == (cpw & 15), NEG, vv)

        prob_buf[...] = jnp.exp(tvals - m_all) / s_tot
        pltpu.sync_copy(prob_buf, out_prob_hbm)

        idx_buf[...] = tidx
        pltpu.async_copy(gps_hbm.at[idx_buf], rows_v, sem).wait()
        pltpu.sync_copy(rows_v, out_gps_hbm)


def _sc_topk(logits, gps_pad):
    mesh = plsc.VectorSubcoreMesh(
        core_axis_name="c", subcore_axis_name="s", num_cores=1)
    f32 = jnp.float32
    return pl.kernel(
        _sc_body,
        out_type=[
            jax.ShapeDtypeStruct((16, 16), f32),
            jax.ShapeDtypeStruct((16,), f32),
        ],
        mesh=mesh,
        scratch_types=[
            pltpu.VMEM((C,), f32),
            pltpu.VMEM((16,), f32),
            pltpu.VMEM((16,), jnp.int32),
            pltpu.VMEM((2, 16), f32),
            pltpu.VMEM((NW * 16,), f32),
            pltpu.VMEM((NW * 16,), jnp.int32),
            pltpu.VMEM((NW, 2, 16), f32),
            pltpu.VMEM((16,), f32),
            pltpu.VMEM((16, 16), f32),
            pltpu.VMEM_SHARED((NW * 16,), f32),
            pltpu.VMEM_SHARED((NW * 16,), jnp.int32),
            pltpu.VMEM_SHARED((NW, 2, 16), f32),
            pltpu.SemaphoreType.DMA,
        ],
        compiler_params=pltpu.CompilerParams(
            needs_layout_passes=False, use_tc_tiling_on_sc=False),
    )(logits, gps_pad)


def kernel(img_feats, top_k, W1, b1, W2, b2, location_feats, gps_gallery,
           logit_scale):
    x0 = img_feats[0:1]
    b1r = b1.reshape(1, -1)
    b2r = b2.reshape(1, -1)
    scale = logit_scale.reshape(1, 1)
    q = _tc_mlp(x0, W1, b1r, W2, b2r, scale)
    logits_sc = _sc_matvec(q.reshape(D_OUT), location_feats)
    logits_tc = _tc_matvec(q, location_feats)
    logits = jnp.concatenate([logits_tc.reshape(KTC), logits_sc])
    gps_pad = jnp.pad(gps_gallery, ((0, 0), (0, 14)))
    out_gps, out_prob = _sc_topk(logits, gps_pad)
    return out_gps[:TOPK, :2], out_prob[:TOPK]


# speedup vs baseline: 1.0036x; 1.0036x over previous
import jax
import jax.numpy as jnp
from jax import lax
from jax.experimental import pallas as pl
from jax.experimental.pallas import tpu as pltpu
from jax.experimental.pallas import tpu_sc as plsc

K = 100000
D_OUT = 512
K_PAD = 100352
TBK = 4096
TNBLK = 18
KTC = TNBLK * TBK
KSC = K_PAD - KTC
RPW = KSC // 32
CH = 16
NPAIR = RPW // CH // 2
NW = 16
C_TC = KTC // NW
C_SC = KSC // NW
VB_T = C_TC // 16
VB_S = C_SC // 16
TOPK = 10
NEG = -1e30
IBIG = 2147483647


def _tc_mlp_body(x_ref, w1_ref, b1_ref, w2_ref, b2_ref, s_ref, q_ref):
    h = jnp.maximum(
        jnp.dot(x_ref[...], w1_ref[...],
                preferred_element_type=jnp.float32) + b1_ref[...], 0.0)
    f = jnp.dot(h, w2_ref[...],
                preferred_element_type=jnp.float32) + b2_ref[...]
    nrm = jnp.maximum(jnp.sqrt(jnp.sum(f * f)), 1e-12)
    q_ref[...] = f * (jnp.exp(s_ref[0, 0]) / nrm)


def _tc_mlp(x, w1, b1, w2, b2, scale):
    return pl.pallas_call(
        _tc_mlp_body,
        out_shape=jax.ShapeDtypeStruct((1, D_OUT), jnp.float32),
    )(x, w1, b1, w2, b2, scale)


def _tc_mv_body(q_ref, loc_ref, out_ref):
    out_ref[...] = lax.dot_general(
        q_ref[...], loc_ref[...], (((1,), (1,)), ((), ())),
        preferred_element_type=jnp.float32)


def _tc_matvec(q, loc):
    return pl.pallas_call(
        _tc_mv_body,
        grid=(TNBLK,),
        in_specs=[
            pl.BlockSpec((1, D_OUT), lambda i: (0, 0)),
            pl.BlockSpec((TBK, D_OUT), lambda i: (i, 0)),
        ],
        out_specs=pl.BlockSpec((1, TBK), lambda i: (0, i)),
        out_shape=jax.ShapeDtypeStruct((1, KTC), jnp.float32),
        compiler_params=pltpu.CompilerParams(
            dimension_semantics=("arbitrary",)),
    )(q, loc)


def _sc_mv_body(q_hbm, loc_hbm, out_hbm, qv, r0, r1, ostage, sem0, sem1):
    cid = lax.axis_index("c")
    sid = lax.axis_index("s")
    w = sid * 2 + cid
    start = KTC + w * RPW
    iot = lax.iota(jnp.int32, 16)

    pltpu.sync_copy(q_hbm, qv)
    qs = [qv[pl.ds(16 * j, 16)] for j in range(32)]

    def row_off(g):
        return jnp.minimum(start + g * CH, K - CH)

    def compute_chunk(rbuf, g):
        o0 = jnp.zeros((16,), jnp.float32)
        for r in range(CH):
            acc = qs[0] * rbuf[r, pl.ds(0, 16)]
            for j in range(1, 32):
                acc = acc + qs[j] * rbuf[r, pl.ds(16 * j, 16)]
            o0 = jnp.where(iot == r, jnp.sum(acc), o0)
        ostage[pl.ds(g * CH, 16)] = o0

    pltpu.async_copy(loc_hbm.at[pl.ds(row_off(0), CH)], r0, sem0)
    pltpu.async_copy(loc_hbm.at[pl.ds(row_off(1), CH)], r1, sem1)

    def body(h, carry):
        g0 = 2 * h
        pltpu.make_async_copy(loc_hbm.at[pl.ds(0, CH)], r0, sem0).wait()
        compute_chunk(r0, g0)

        @pl.when(h < NPAIR - 1)
        def _():
            pltpu.async_copy(loc_hbm.at[pl.ds(row_off(g0 + 2), CH)], r0, sem0)

        pltpu.make_async_copy(loc_hbm.at[pl.ds(0, CH)], r1, sem1).wait()
        compute_chunk(r1, g0 + 1)

        @pl.when(h < NPAIR - 1)
        def _():
            pltpu.async_copy(loc_hbm.at[pl.ds(row_off(g0 + 3), CH)], r1, sem1)

        return carry

    lax.fori_loop(0, NPAIR, body, 0)
    pltpu.sync_copy(ostage, out_hbm.at[pl.ds(w * RPW, RPW)])


def _sc_matvec(q, loc):
    mesh = plsc.VectorSubcoreMesh(core_axis_name="c", subcore_axis_name="s")
    f32 = jnp.float32
    return pl.kernel(
        _sc_mv_body,
        out_type=jax.ShapeDtypeStruct((KSC,), f32),
        mesh=mesh,
        scratch_types=[
            pltpu.VMEM((D_OUT,), f32),
            pltpu.VMEM((CH, D_OUT), f32),
            pltpu.VMEM((CH, D_OUT), f32),
            pltpu.VMEM((RPW,), f32),
            pltpu.SemaphoreType.DMA,
            pltpu.SemaphoreType.DMA,
        ],
        compiler_params=pltpu.CompilerParams(
            needs_layout_passes=False, use_tc_tiling_on_sc=False),
    )(q, loc)


def _sc_body(ltc_hbm, lsc_hbm, gps_hbm, out_gps_hbm, out_prob_hbm,
             buf_t, buf_s, vals_buf, idx_buf, ms_buf,
             mvals, midx, mms, prob_buf, rows_v,
             sh_vals, sh_idx, sh_ms, sem):
    wid = lax.axis_index("s")
    base_t = wid * C_TC
    base_s = KTC + wid * C_SC
    iot = lax.iota(jnp.int32, 16)

    pltpu.sync_copy(ltc_hbm.at[pl.ds(base_t, C_TC)], buf_t)
    pltpu.sync_copy(lsc_hbm.at[pl.ds(wid * C_SC, C_SC)], buf_s)

    def p_max_t(j, m_vec):
        return jnp.maximum(m_vec, buf_t[pl.ds(j * 16, 16)])

    def p_mask_s(j, m_vec):
        v = buf_s[pl.ds(j * 16, 16)]
        gidx = base_s + j * 16 + iot
        v = jnp.where(gidx < K, v, NEG)
        buf_s[pl.ds(j * 16, 16)] = v
        return jnp.maximum(m_vec, v)

    m_vec = lax.fori_loop(0, VB_T, p_max_t,
                          jnp.full((16,), NEG, jnp.float32))
    m_vec = lax.fori_loop(0, VB_S, p_mask_s, m_vec)
    m_w = jnp.max(m_vec)

    def p_sum_t(j, s_vec):
        return s_vec + jnp.exp(buf_t[pl.ds(j * 16, 16)] - m_w)

    def p_sum_s(j, s_vec):
        return s_vec + jnp.exp(buf_s[pl.ds(j * 16, 16)] - m_w)

    s_vec = lax.fori_loop(0, VB_T, p_sum_t, jnp.zeros((16,), jnp.float32))
    s_vec = lax.fori_loop(0, VB_S, p_sum_s, s_vec)

    vals_vec = jnp.full((16,), NEG, jnp.float32)
    idx_vec = jnp.zeros((16,), jnp.int32)
    for i in range(TOPK):
        def p_top_t(j, carry):
            mx, mi = carry
            v = buf_t[pl.ds(j * 16, 16)]
            gidx = base_t + j * 16 + iot
            c = v > mx
            return jnp.where(c, v, mx), jnp.where(c, gidx, mi)

        def p_top_s(j, carry):
            mx, mi = carry
            v = buf_s[pl.ds(j * 16, 16)]
            gidx = base_s + j * 16 + iot
            c = v > mx
            return jnp.where(c, v, mx), jnp.where(c, gidx, mi)

        mx, mi = lax.fori_loop(
            0, VB_T, p_top_t,
            (jnp.full((16,), NEG, jnp.float32), jnp.zeros((16,), jnp.int32)))
        mx, mi = lax.fori_loop(0, VB_S, p_top_s, (mx, mi))
        gm = jnp.max(mx)
        gi = jnp.min(jnp.where(mx == gm, mi, IBIG))
        vals_vec = jnp.where(iot == i, gm, vals_vec)
        idx_vec = jnp.where(iot == i, gi, idx_vec)

        in_t = gi < KTC
        lo_t = jnp.where(in_t, gi - base_t, 0)
        lo_s = jnp.where(in_t, 0, gi - base_s)
        j0t = lo_t & ~15
        j0s = lo_s & ~15
        vvt = buf_t[pl.ds(j0t, 16)]
        buf_t[pl.ds(j0t, 16)] = jnp.where(
            (iot == (lo_t & 15)) & in_t, NEG, vvt)
        vvs = buf_s[pl.ds(j0s, 16)]
        buf_s[pl.ds(j0s, 16)] = jnp.where(
            (iot == (lo_s & 15)) & jnp.logical_not(in_t), NEG, vvs)

    vals_buf[...] = vals_vec
    idx_buf[...] = idx_vec
    ms_buf[0, :] = jnp.broadcast_to(m_w, (16,))
    ms_buf[1, :] = s_vec

    pltpu.sync_copy(vals_buf, sh_vals.at[pl.ds(wid * 16, 16)])
    pltpu.sync_copy(idx_buf, sh_idx.at[pl.ds(wid * 16, 16)])
    pltpu.sync_copy(ms_buf, sh_ms.at[wid])
    plsc.subcore_barrier()

    @pl.when(wid == 0)
    def _merge():
        pltpu.sync_copy(sh_vals, mvals)
        pltpu.sync_copy(sh_idx, midx)
        pltpu.sync_copy(sh_ms, mms)

        m_all = jnp.full((16,), NEG, jnp.float32)
        for w in range(NW):
            m_all = jnp.maximum(m_all, mms[w, 0, :])
        s_all = jnp.zeros((16,), jnp.float32)
        for w in range(NW):
            s_all = s_all + mms[w, 1, :] * jnp.exp(mms[w, 0, :] - m_all)
        s_tot = jnp.sum(s_all)

        tvals = jnp.full((16,), NEG, jnp.float32)
        tidx = jnp.zeros((16,), jnp.int32)
        for i in range(TOPK):
            mx = jnp.full((16,), NEG, jnp.float32)
            gx = jnp.zeros((16,), jnp.int32)
            cp = jnp.zeros((16,), jnp.int32)
            for w in range(NW):
                v = mvals[pl.ds(w * 16, 16)]
                c = v > mx
                mx = jnp.where(c, v, mx)
                gx = jnp.where(c, midx[pl.ds(w * 16, 16)], gx)
                cp = jnp.where(c, w * 16 + iot, cp)
            gm = jnp.max(mx)
            gi = jnp.min(jnp.where(mx == gm, gx, IBIG))
            cpw = jnp.min(jnp.where((mx == gm) & (gx == gi), cp, IBIG))
            tvals = jnp.where(iot == i, gm, tvals)
            tidx = jnp.where(iot == i, gi, tidx)
            j0 = cpw & ~15
            vv = mvals[pl.ds(j0, 16)]
            mvals[pl.ds(j0, 16)] = jnp.where(iot == (cpw & 15), NEG, vv)

        prob_buf[...] = jnp.exp(tvals - m_all) / s_tot
        pltpu.sync_copy(prob_buf, out_prob_hbm)

        idx_buf[...] = tidx
        pltpu.async_copy(gps_hbm.at[idx_buf], rows_v, sem).wait()
        pltpu.sync_copy(rows_v, out_gps_hbm)


def _sc_topk(logits_tc, logits_sc, gps_pad):
    mesh = plsc.VectorSubcoreMesh(
        core_axis_name="c", subcore_axis_name="s", num_cores=1)
    f32 = jnp.float32
    return pl.kernel(
        _sc_body,
        out_type=[
            jax.ShapeDtypeStruct((16, 16), f32),
            jax.ShapeDtypeStruct((16,), f32),
        ],
        mesh=mesh,
        scratch_types=[
            pltpu.VMEM((C_TC,), f32),
            pltpu.VMEM((C_SC,), f32),
            pltpu.VMEM((16,), f32),
            pltpu.VMEM((16,), jnp.int32),
            pltpu.VMEM((2, 16), f32),
            pltpu.VMEM((NW * 16,), f32),
            pltpu.VMEM((NW * 16,), jnp.int32),
            pltpu.VMEM((NW, 2, 16), f32),
            pltpu.VMEM((16,), f32),
            pltpu.VMEM((16, 16), f32),
            pltpu.VMEM_SHARED((NW * 16,), f32),
            pltpu.VMEM_SHARED((NW * 16,), jnp.int32),
            pltpu.VMEM_SHARED((NW, 2, 16), f32),
            pltpu.SemaphoreType.DMA,
        ],
        compiler_params=pltpu.CompilerParams(
            needs_layout_passes=False, use_tc_tiling_on_sc=False),
    )(logits_tc, logits_sc, gps_pad)


def kernel(img_feats, top_k, W1, b1, W2, b2, location_feats, gps_gallery,
           logit_scale):
    x0 = img_feats[0:1]
    b1r = b1.reshape(1, -1)
    b2r = b2.reshape(1, -1)
    scale = logit_scale.reshape(1, 1)
    q = _tc_mlp(x0, W1, b1r, W2, b2r, scale)
    logits_sc = _sc_matvec(q.reshape(D_OUT), location_feats)
    logits_tc = _tc_matvec(q, location_feats)
    gps_pad = jnp.pad(gps_gallery, ((0, 0), (0, 14)))
    out_gps, out_prob = _sc_topk(logits_tc.reshape(KTC), logits_sc, gps_pad)
    return out_gps[:TOPK, :2], out_prob[:TOPK]


# speedup vs baseline: 1.7046x; 1.6986x over previous
import jax
import jax.numpy as jnp
from jax import lax
from jax.experimental import pallas as pl
from jax.experimental.pallas import tpu as pltpu
from jax.experimental.pallas import tpu_sc as plsc

K = 100000
D_OUT = 512
K_PAD = 100352
TBK = 4096
TNBLK = 18
KTC = TNBLK * TBK
KSC = K_PAD - KTC
RPW = KSC // 32
CH = 16
NPAIR = RPW // CH // 2
NW = 16
C_TC = KTC // NW
C_SC = KSC // NW
VB_T = C_TC // 16
VB_S = C_SC // 16
TOPK = 10
NEG = -1e30
IBIG = 2147483647


def _tc_mlp_body(x_ref, w1_ref, b1_ref, w2_ref, b2_ref, s_ref, q_ref):
    h = jnp.maximum(
        jnp.dot(x_ref[...], w1_ref[...],
                preferred_element_type=jnp.float32) + b1_ref[...], 0.0)
    f = jnp.dot(h, w2_ref[...],
                preferred_element_type=jnp.float32) + b2_ref[...]
    nrm = jnp.maximum(jnp.sqrt(jnp.sum(f * f)), 1e-12)
    q_ref[...] = f * (jnp.exp(s_ref[0, 0]) / nrm)


def _tc_mlp(x, w1, b1, w2, b2, scale):
    return pl.pallas_call(
        _tc_mlp_body,
        out_shape=jax.ShapeDtypeStruct((1, D_OUT), jnp.float32),
    )(x, w1, b1, w2, b2, scale)


def _tc_mv_body(q_ref, loc_ref, out_ref):
    out_ref[...] = lax.dot_general(
        q_ref[...], loc_ref[...], (((1,), (1,)), ((), ())),
        preferred_element_type=jnp.float32)


def _tc_matvec(q, loc):
    return pl.pallas_call(
        _tc_mv_body,
        grid=(TNBLK,),
        in_specs=[
            pl.BlockSpec((1, D_OUT), lambda i: (0, 0)),
            pl.BlockSpec((TBK, D_OUT), lambda i: (i, 0)),
        ],
        out_specs=pl.BlockSpec((1, TBK), lambda i: (0, i)),
        out_shape=jax.ShapeDtypeStruct((1, KTC), jnp.float32),
        compiler_params=pltpu.CompilerParams(
            dimension_semantics=("arbitrary",)),
    )(q, loc)


def _sc_mv_body(q_hbm, loc_hbm, out_hbm, qv, r0, r1, ostage, sem0, sem1):
    cid = lax.axis_index("c")
    sid = lax.axis_index("s")
    w = sid * 2 + cid
    start = KTC + w * RPW
    iot = lax.iota(jnp.int32, 16)

    pltpu.sync_copy(q_hbm, qv)
    qs = [qv[pl.ds(16 * j, 16)] for j in range(32)]

    def row_off(g):
        return jnp.minimum(start + g * CH, K - CH)

    def compute_chunk(rbuf, g):
        o0 = jnp.zeros((16,), jnp.float32)
        for r in range(CH):
            acc = qs[0] * rbuf[r, pl.ds(0, 16)]
            for j in range(1, 32):
                acc = acc + qs[j] * rbuf[r, pl.ds(16 * j, 16)]
            o0 = jnp.where(iot == r, jnp.sum(acc), o0)
        ostage[pl.ds(g * CH, 16)] = o0

    pltpu.async_copy(loc_hbm.at[pl.ds(row_off(0), CH)], r0, sem0)
    pltpu.async_copy(loc_hbm.at[pl.ds(row_off(1), CH)], r1, sem1)

    def body(h, carry):
        g0 = 2 * h
        pltpu.make_async_copy(loc_hbm.at[pl.ds(0, CH)], r0, sem0).wait()
        compute_chunk(r0, g0)

        @pl.when(h < NPAIR - 1)
        def _():
            pltpu.async_copy(loc_hbm.at[pl.ds(row_off(g0 + 2), CH)], r0, sem0)

        pltpu.make_async_copy(loc_hbm.at[pl.ds(0, CH)], r1, sem1).wait()
        compute_chunk(r1, g0 + 1)

        @pl.when(h < NPAIR - 1)
        def _():
            pltpu.async_copy(loc_hbm.at[pl.ds(row_off(g0 + 3), CH)], r1, sem1)

        return carry

    lax.fori_loop(0, NPAIR, body, 0)
    pltpu.sync_copy(ostage, out_hbm.at[pl.ds(w * RPW, RPW)])


def _sc_matvec(q, loc):
    mesh = plsc.VectorSubcoreMesh(core_axis_name="c", subcore_axis_name="s")
    f32 = jnp.float32
    return pl.kernel(
        _sc_mv_body,
        out_type=jax.ShapeDtypeStruct((KSC,), f32),
        mesh=mesh,
        scratch_types=[
            pltpu.VMEM((D_OUT,), f32),
            pltpu.VMEM((CH, D_OUT), f32),
            pltpu.VMEM((CH, D_OUT), f32),
            pltpu.VMEM((RPW,), f32),
            pltpu.SemaphoreType.DMA,
            pltpu.SemaphoreType.DMA,
        ],
        compiler_params=pltpu.CompilerParams(
            needs_layout_passes=False, use_tc_tiling_on_sc=True),
    )(q, loc)


def _sc_body(ltc_hbm, lsc_hbm, gps_hbm, out_gps_hbm, out_prob_hbm,
             buf_t, buf_s, vals_buf, idx_buf, ms_buf,
             mvals, midx, mms, prob_buf, rows_v,
             sh_vals, sh_idx, sh_ms, sem):
    wid = lax.axis_index("s")
    base_t = wid * C_TC
    base_s = KTC + wid * C_SC
    iot = lax.iota(jnp.int32, 16)

    pltpu.sync_copy(ltc_hbm.at[pl.ds(base_t, C_TC)], buf_t)
    pltpu.sync_copy(lsc_hbm.at[pl.ds(wid * C_SC, C_SC)], buf_s)

    def p_max_t(j, m_vec):
        return jnp.maximum(m_vec, buf_t[pl.ds(j * 16, 16)])

    def p_mask_s(j, m_vec):
        v = buf_s[pl.ds(j * 16, 16)]
        gidx = base_s + j * 16 + iot
        v = jnp.where(gidx < K, v, NEG)
        buf_s[pl.ds(j * 16, 16)] = v
        return jnp.maximum(m_vec, v)

    m_vec = lax.fori_loop(0, VB_T, p_max_t,
                          jnp.full((16,), NEG, jnp.float32))
    m_vec = lax.fori_loop(0, VB_S, p_mask_s, m_vec)
    m_w = jnp.max(m_vec)

    def p_sum_t(j, s_vec):
        return s_vec + jnp.exp(buf_t[pl.ds(j * 16, 16)] - m_w)

    def p_sum_s(j, s_vec):
        return s_vec + jnp.exp(buf_s[pl.ds(j * 16, 16)] - m_w)

    s_vec = lax.fori_loop(0, VB_T, p_sum_t, jnp.zeros((16,), jnp.float32))
    s_vec = lax.fori_loop(0, VB_S, p_sum_s, s_vec)

    vals_vec = jnp.full((16,), NEG, jnp.float32)
    idx_vec = jnp.zeros((16,), jnp.int32)
    for i in range(TOPK):
        def p_top_t(j, carry):
            mx, mi = carry
            v = buf_t[pl.ds(j * 16, 16)]
            gidx = base_t + j * 16 + iot
            c = v > mx
            return jnp.where(c, v, mx), jnp.where(c, gidx, mi)

        def p_top_s(j, carry):
            mx, mi = carry
            v = buf_s[pl.ds(j * 16, 16)]
            gidx = base_s + j * 16 + iot
            c = v > mx
            return jnp.where(c, v, mx), jnp.where(c, gidx, mi)

        mx, mi = lax.fori_loop(
            0, VB_T, p_top_t,
            (jnp.full((16,), NEG, jnp.float32), jnp.zeros((16,), jnp.int32)))
        mx, mi = lax.fori_loop(0, VB_S, p_top_s, (mx, mi))
        gm = jnp.max(mx)
        gi = jnp.min(jnp.where(mx == gm, mi, IBIG))
        vals_vec = jnp.where(iot == i, gm, vals_vec)
        idx_vec = jnp.where(iot == i, gi, idx_vec)

        in_t = gi < KTC
        lo_t = jnp.where(in_t, gi - base_t, 0)
        lo_s = jnp.where(in_t, 0, gi - base_s)
        j0t = lo_t & ~15
        j0s = lo_s & ~15
        vvt = buf_t[pl.ds(j0t, 16)]
        buf_t[pl.ds(j0t, 16)] = jnp.where(
            (iot == (lo_t & 15)) & in_t, NEG, vvt)
        vvs = buf_s[pl.ds(j0s, 16)]
        buf_s[pl.ds(j0s, 16)] = jnp.where(
            (iot == (lo_s & 15)) & jnp.logical_not(in_t), NEG, vvs)

    vals_buf[...] = vals_vec
    idx_buf[...] = idx_vec
    ms_buf[0, :] = jnp.broadcast_to(m_w, (16,))
    ms_buf[1, :] = s_vec

    pltpu.sync_copy(vals_buf, sh_vals.at[pl.ds(wid * 16, 16)])
    pltpu.sync_copy(idx_buf, sh_idx.at[pl.ds(wid * 16, 16)])
    pltpu.sync_copy(ms_buf, sh_ms.at[wid])
    plsc.subcore_barrier()

    @pl.when(wid == 0)
    def _merge():
        pltpu.sync_copy(sh_vals, mvals)
        pltpu.sync_copy(sh_idx, midx)
        pltpu.sync_copy(sh_ms, mms)

        m_all = jnp.full((16,), NEG, jnp.float32)
        for w in range(NW):
            m_all = jnp.maximum(m_all, mms[w, 0, :])
        s_all = jnp.zeros((16,), jnp.float32)
        for w in range(NW):
            s_all = s_all + mms[w, 1, :] * jnp.exp(mms[w, 0, :] - m_all)
        s_tot = jnp.sum(s_all)

        tvals = jnp.full((16,), NEG, jnp.float32)
        tidx = jnp.zeros((16,), jnp.int32)
        for i in range(TOPK):
            mx = jnp.full((16,), NEG, jnp.float32)
            gx = jnp.zeros((16,), jnp.int32)
            cp = jnp.zeros((16,), jnp.int32)
            for w in range(NW):
                v = mvals[pl.ds(w * 16, 16)]
                c = v > mx
                mx = jnp.where(c, v, mx)
                gx = jnp.where(c, midx[pl.ds(w * 16, 16)], gx)
                cp = jnp.where(c, w * 16 + iot, cp)
            gm = jnp.max(mx)
            gi = jnp.min(jnp.where(mx == gm, gx, IBIG))
            cpw = jnp.min(jnp.where((mx == gm) & (gx == gi), cp, IBIG))
            tvals = jnp.where(iot == i, gm, tvals)
            tidx = jnp.where(iot == i, gi, tidx)
            j0 = cpw & ~15
            vv = mvals[pl.ds(j0, 16)]
            mvals[pl.ds(j0, 16)] = jnp.where(iot == (cpw & 15), NEG, vv)

        prob_buf[...] = jnp.exp(tvals - m_all) / s_tot
        pltpu.sync_copy(prob_buf, out_prob_hbm)

        idx_buf[...] = tidx
        pltpu.async_copy(gps_hbm.at[idx_buf], rows_v, sem).wait()
        pltpu.sync_copy(rows_v, out_gps_hbm)


def _sc_topk(logits_tc, logits_sc, gps_pad):
    mesh = plsc.VectorSubcoreMesh(
        core_axis_name="c", subcore_axis_name="s", num_cores=1)
    f32 = jnp.float32
    return pl.kernel(
        _sc_body,
        out_type=[
            jax.ShapeDtypeStruct((16, 16), f32),
            jax.ShapeDtypeStruct((16,), f32),
        ],
        mesh=mesh,
        scratch_types=[
            pltpu.VMEM((C_TC,), f32),
            pltpu.VMEM((C_SC,), f32),
            pltpu.VMEM((16,), f32),
            pltpu.VMEM((16,), jnp.int32),
            pltpu.VMEM((2, 16), f32),
            pltpu.VMEM((NW * 16,), f32),
            pltpu.VMEM((NW * 16,), jnp.int32),
            pltpu.VMEM((NW, 2, 16), f32),
            pltpu.VMEM((16,), f32),
            pltpu.VMEM((16, 16), f32),
            pltpu.VMEM_SHARED((NW * 16,), f32),
            pltpu.VMEM_SHARED((NW * 16,), jnp.int32),
            pltpu.VMEM_SHARED((NW, 2, 16), f32),
            pltpu.SemaphoreType.DMA,
        ],
        compiler_params=pltpu.CompilerParams(
            needs_layout_passes=False, use_tc_tiling_on_sc=False),
    )(logits_tc, logits_sc, gps_pad)


def kernel(img_feats, top_k, W1, b1, W2, b2, location_feats, gps_gallery,
           logit_scale):
    x0 = img_feats[0:1]
    b1r = b1.reshape(1, -1)
    b2r = b2.reshape(1, -1)
    scale = logit_scale.reshape(1, 1)
    q = _tc_mlp(x0, W1, b1r, W2, b2r, scale)
    logits_sc = _sc_matvec(q.reshape(D_OUT), location_feats)
    logits_tc = _tc_matvec(q, location_feats)
    gps_pad = jnp.pad(gps_gallery, ((0, 0), (0, 14)))
    out_gps, out_prob = _sc_topk(logits_tc.reshape(KTC), logits_sc, gps_pad)
    return out_gps[:TOPK, :2], out_prob[:TOPK]


# speedup vs baseline: 1.7285x; 1.0140x over previous
import jax
import jax.numpy as jnp
from jax import lax
from jax.experimental import pallas as pl
from jax.experimental.pallas import tpu as pltpu
from jax.experimental.pallas import tpu_sc as plsc

K = 100000
D_OUT = 512
BK = 4096
NBLK = 25
K_PAD = NBLK * BK
NW = 16
C = K_PAD // NW
VB = C // 16
TOPK = 10
NEG = -1e30
IBIG = 2147483647


def _tc_mv_body(q_ref, s_ref, loc_ref, out_ref):
    d = lax.dot_general(
        q_ref[...], loc_ref[...], (((1,), (1,)), ((), ())),
        preferred_element_type=jnp.float32)
    out_ref[...] = jnp.exp(s_ref[0, 0]) * d


def _tc_matvec(q, scale, loc):
    return pl.pallas_call(
        _tc_mv_body,
        grid=(NBLK,),
        in_specs=[
            pl.BlockSpec((1, D_OUT), lambda i: (0, 0)),
            pl.BlockSpec((1, 1), lambda i: (0, 0)),
            pl.BlockSpec((BK, D_OUT), lambda i: (i, 0)),
        ],
        out_specs=pl.BlockSpec((1, BK), lambda i: (0, i)),
        out_shape=jax.ShapeDtypeStruct((1, K_PAD), jnp.float32),
        compiler_params=pltpu.CompilerParams(
            dimension_semantics=("arbitrary",)),
    )(q, scale, loc)


def _sc_body(logits_hbm, gps_hbm, out_gps_hbm, out_prob_hbm,
             buf, vals_buf, idx_buf, ms_buf,
             mvals, midx, mms, prob_buf, rows_v,
             sh_vals, sh_idx, sh_ms, sem):
    wid = lax.axis_index("s")
    base = wid * C
    iot = lax.iota(jnp.int32, 16)

    pltpu.sync_copy(logits_hbm.at[pl.ds(base, C)], buf)

    def p_mask(j, m_vec):
        v = buf[pl.ds(j * 16, 16)]
        gidx = base + j * 16 + iot
        v = jnp.where(gidx < K, v, NEG)
        buf[pl.ds(j * 16, 16)] = v
        return jnp.maximum(m_vec, v)

    m_vec = lax.fori_loop(0, VB, p_mask, jnp.full((16,), NEG, jnp.float32))
    m_w = jnp.max(m_vec)

    def p_sum(j, s_vec):
        v = buf[pl.ds(j * 16, 16)]
        return s_vec + jnp.exp(v - m_w)

    s_vec = lax.fori_loop(0, VB, p_sum, jnp.zeros((16,), jnp.float32))

    vals_vec = jnp.full((16,), NEG, jnp.float32)
    idx_vec = jnp.zeros((16,), jnp.int32)
    for i in range(TOPK):
        def p_top(j, carry):
            mx, mi = carry
            v = buf[pl.ds(j * 16, 16)]
            gidx = base + j * 16 + iot
            c = v > mx
            return jnp.where(c, v, mx), jnp.where(c, gidx, mi)

        mx, mi = lax.fori_loop(
            0, VB, p_top,
            (jnp.full((16,), NEG, jnp.float32), jnp.zeros((16,), jnp.int32)))
        gm = jnp.max(mx)
        gi = jnp.min(jnp.where(mx == gm, mi, IBIG))
        vals_vec = jnp.where(iot == i, gm, vals_vec)
        idx_vec = jnp.where(iot == i, gi, idx_vec)
        lo = gi - base
        j0 = lo & ~15
        vv = buf[pl.ds(j0, 16)]
        buf[pl.ds(j0, 16)] = jnp.where(iot == (lo & 15), NEG, vv)

    vals_buf[...] = vals_vec
    idx_buf[...] = idx_vec
    ms_buf[0, :] = jnp.broadcast_to(m_w, (16,))
    ms_buf[1, :] = s_vec

    pltpu.sync_copy(vals_buf, sh_vals.at[pl.ds(wid * 16, 16)])
    pltpu.sync_copy(idx_buf, sh_idx.at[pl.ds(wid * 16, 16)])
    pltpu.sync_copy(ms_buf, sh_ms.at[wid])
    plsc.subcore_barrier()

    @pl.when(wid == 0)
    def _merge():
        pltpu.sync_copy(sh_vals, mvals)
        pltpu.sync_copy(sh_idx, midx)
        pltpu.sync_copy(sh_ms, mms)

        m_all = jnp.full((16,), NEG, jnp.float32)
        for w in range(NW):
            m_all = jnp.maximum(m_all, mms[w, 0, :])
        s_all = jnp.zeros((16,), jnp.float32)
        for w in range(NW):
            s_all = s_all + mms[w, 1, :] * jnp.exp(mms[w, 0, :] - m_all)
        s_tot = jnp.sum(s_all)

        tvals = jnp.full((16,), NEG, jnp.float32)
        tidx = jnp.zeros((16,), jnp.int32)
        for i in range(TOPK):
            mx = jnp.full((16,), NEG, jnp.float32)
            gx = jnp.zeros((16,), jnp.int32)
            cp = jnp.zeros((16,), jnp.int32)
            for w in range(NW):
                v = mvals[pl.ds(w * 16, 16)]
                c = v > mx
                mx = jnp.where(c, v, mx)
                gx = jnp.where(c, midx[pl.ds(w * 16, 16)], gx)
                cp = jnp.where(c, w * 16 + iot, cp)
            gm = jnp.max(mx)
            gi = jnp.min(jnp.where(mx == gm, gx, IBIG))
            cpw = jnp.min(jnp.where((mx == gm) & (gx == gi), cp, IBIG))
            tvals = jnp.where(iot == i, gm, tvals)
            tidx = jnp.where(iot == i, gi, tidx)
            j0 = cpw & ~15
            vv = mvals[pl.ds(j0, 16)]
            mvals[pl.ds(j0, 16)] = jnp.where(iot == (cpw & 15), NEG, vv)

        prob_buf[...] = jnp.exp(tvals - m_all) / s_tot
        pltpu.sync_copy(prob_buf, out_prob_hbm)

        idx_buf[...] = tidx
        pltpu.async_copy(gps_hbm.at[idx_buf], rows_v, sem).wait()
        pltpu.sync_copy(rows_v, out_gps_hbm)


def _sc_topk(logits, gps_pad):
    mesh = plsc.VectorSubcoreMesh(
        core_axis_name="c", subcore_axis_name="s", num_cores=1)
    f32 = jnp.float32
    return pl.kernel(
        _sc_body,
        out_type=[
            jax.ShapeDtypeStruct((16, 16), f32),
            jax.ShapeDtypeStruct((16,), f32),
        ],
        mesh=mesh,
        scratch_types=[
            pltpu.VMEM((C,), f32),
            pltpu.VMEM((16,), f32),
            pltpu.VMEM((16,), jnp.int32),
            pltpu.VMEM((2, 16), f32),
            pltpu.VMEM((NW * 16,), f32),
            pltpu.VMEM((NW * 16,), jnp.int32),
            pltpu.VMEM((NW, 2, 16), f32),
            pltpu.VMEM((16,), f32),
            pltpu.VMEM((16, 16), f32),
            pltpu.VMEM_SHARED((NW * 16,), f32),
            pltpu.VMEM_SHARED((NW * 16,), jnp.int32),
            pltpu.VMEM_SHARED((NW, 2, 16), f32),
            pltpu.SemaphoreType.DMA,
        ],
        compiler_params=pltpu.CompilerParams(
            needs_layout_passes=False, use_tc_tiling_on_sc=False),
    )(logits, gps_pad)


def kernel(img_feats, top_k, W1, b1, W2, b2, location_feats, gps_gallery,
           logit_scale):
    h = jax.nn.relu(img_feats @ W1 + b1)
    f = h @ W2 + b2
    f = f / jnp.maximum(jnp.linalg.norm(f, axis=1, keepdims=True), 1e-12)
    q = f[0:1]

    logits = _tc_matvec(q, logit_scale.reshape(1, 1), location_feats)
    gps_pad = jnp.pad(gps_gallery, ((0, 0), (0, 14)))
    out_gps, out_prob = _sc_topk(logits.reshape(K_PAD), gps_pad)
    return out_gps[:TOPK, :2], out_prob[:TOPK]
